# baseline (speedup 1.0000x reference)
import functools

import jax
import jax.numpy as jnp
from jax import lax
from jax.experimental import pallas as pl
from jax.experimental.pallas import tpu as pltpu
from jax.experimental.pallas import tpu_sc as plsc

_VOCAB = 1000000
_DIM = 64
_BATCH = 16384
_NC = 2
_NS = 16
_NW = _NC * _NS
_BPW = _BATCH // _NW
_CHUNK = 128
_NCHUNK = _BPW // _CHUNK


def _reg_gather(x, idx):
    dnums = lax.GatherDimensionNumbers(
        offset_dims=(), collapsed_slice_dims=(0,), start_index_map=(0,))
    return lax.gather(x, idx[:, None], dnums, (1,),
                      mode=lax.GatherScatterMode.PROMISE_IN_BOUNDS)


def _f16_bits(x):
    b = plsc.bitcast(x, jnp.int32)
    sign = lax.shift_right_logical(b, 16) & 0x8000
    mag = b & 0x7FFFFFFF
    rnd = mag + 0xFFF + (lax.shift_right_logical(mag, 13) & 1)
    h = lax.shift_right_logical(rnd, 13) - (112 << 10)
    return sign | jnp.maximum(h, 0)


def _dequant_body(idx_hbm, w_hbm, s_hbm, out_hbm, idx_v, rows_v, sc_v, out_v, sem):
    wid = lax.axis_index("s") * _NC + lax.axis_index("c")
    base = wid * _BPW

    pltpu.sync_copy(idx_hbm.at[pl.ds(base, _BPW)], idx_v)
    copies = []
    for j in range(_NCHUNK):
        isl = idx_v.at[pl.ds(j * _CHUNK, _CHUNK)]
        copies.append(pltpu.async_copy(
            w_hbm.at[isl], rows_v.at[pl.ds(j * _CHUNK, _CHUNK)], sem))
        copies.append(pltpu.async_copy(
            s_hbm.at[isl], sc_v.at[pl.ds(j * _CHUNK, _CHUNK)], sem))
    for c in copies:
        c.wait()

    lane = lax.iota(jnp.int32, 16)
    src_lane = lane >> 1
    lsh_e = 24 - 16 * (lane & 1)
    lsh_o = 16 - 16 * (lane & 1)

    def blk(t, _):
        sp16 = sc_v[pl.ds(t * 16, 16)]
        s0v = plsc.bitcast(((sp16 & 0x7FFF) << 13) + 0x38000000, jnp.float32)
        s1v = plsc.bitcast((((sp16 >> 16) & 0x7FFF) << 13) + 0x38000000,
                           jnp.float32)
        for r in range(16):
            i = t * 16 + r
            row32 = rows_v[i]
            s0 = jnp.broadcast_to(s0v[r], (16,))
            s1 = jnp.broadcast_to(s1v[r], (16,))
            for h, sv in ((0, s0), (1, s1)):
                g = _reg_gather(row32, src_lane + 8 * h)
                e = ((g << lsh_e) >> 24).astype(jnp.float32) * sv
                o = ((g << lsh_o) >> 24).astype(jnp.float32) * sv
                word = _f16_bits(e) | (_f16_bits(o) << 16)
                out_v[i, pl.ds(16 * h, 16)] = word
        return 0

    lax.fori_loop(0, _BPW // 16, blk, 0)
    pltpu.sync_copy(out_v, out_hbm.at[pl.ds(base, _BPW)])


_mesh = plsc.VectorSubcoreMesh(core_axis_name="c", subcore_axis_name="s")

_sc_lookup = functools.partial(
    pl.kernel,
    out_type=jax.ShapeDtypeStruct((_BATCH, _DIM // 2), jnp.int32),
    mesh=_mesh,
    scratch_types=[
        pltpu.VMEM((_BPW,), jnp.int32),
        pltpu.VMEM((_BPW, _DIM // 4), jnp.int32),
        pltpu.VMEM((_BPW,), jnp.int32),
        pltpu.VMEM((_BPW, _DIM // 2), jnp.int32),
        pltpu.SemaphoreType.DMA,
    ],
    compiler_params=pltpu.CompilerParams(
        needs_layout_passes=False, use_tc_tiling_on_sc=False),
)(_dequant_body)


def kernel(indices, weight, scales):
    idx = indices.astype(jnp.int32)
    w_packed = lax.bitcast_convert_type(
        weight.reshape(_VOCAB, _DIM // 4, 4), jnp.int32)
    s_packed = lax.bitcast_convert_type(scales, jnp.int32)
    out32 = _sc_lookup(idx, w_packed, s_packed)
    return lax.bitcast_convert_type(out32, jnp.float16).reshape(_BATCH, _DIM)


# speedup vs baseline: 1.0095x; 1.0095x over previous
import functools

import jax
import jax.numpy as jnp
from jax import lax
from jax.experimental import pallas as pl
from jax.experimental.pallas import tpu as pltpu
from jax.experimental.pallas import tpu_sc as plsc

_VOCAB = 1000000
_DIM = 64
_BATCH = 16384
_NC = 2
_NS = 16
_NW = _NC * _NS
_BPW = _BATCH // _NW
_CHUNK = 128
_NCHUNK = _BPW // _CHUNK


def _f16_bits(x):
    b = plsc.bitcast(x, jnp.int32)
    sign = lax.shift_right_logical(b, 16) & 0x8000
    mag = b & 0x7FFFFFFF
    rnd = mag + 0xFFF + (lax.shift_right_logical(mag, 13) & 1)
    h = lax.shift_right_logical(rnd, 13) - (112 << 10)
    return sign | jnp.maximum(h, 0)


def _dequant_body(idx_hbm, w_hbm, s_hbm, out_hbm,
                  idx_v, rows_v, sc_v, stage_v, sem):
    wid = lax.axis_index("s") * _NC + lax.axis_index("c")
    base = wid * _BPW

    pltpu.sync_copy(idx_hbm.at[pl.ds(base, _BPW)], idx_v)
    copies = []
    for q in range(_NCHUNK):
        sl = pl.ds(q * _CHUNK, _CHUNK)
        isl = idx_v.at[sl]
        copies.append(pltpu.async_copy(w_hbm.at[isl], rows_v.at[sl], sem))
        copies.append(pltpu.async_copy(s_hbm.at[isl], sc_v.at[sl], sem))
    for c in copies:
        c.wait()

    lane = lax.iota(jnp.int32, 16)

    def blk(t, _):
        rs = pl.ds(t * 16, 16)
        rows16 = t * 16 + lane
        sp = sc_v[rs]
        s0v = plsc.bitcast(((sp & 0x7FFF) << 13) + 0x38000000, jnp.float32)
        s1v = plsc.bitcast((((sp >> 16) & 0x7FFF) << 13) + 0x38000000,
                           jnp.float32)
        for j in range(16):
            v = plsc.load_gather(rows_v, [rows16, lane * 0 + j])
            sv = s0v if j < 8 else s1v
            f0 = ((v << 24) >> 24).astype(jnp.float32) * sv
            f1 = ((v << 16) >> 24).astype(jnp.float32) * sv
            f2 = ((v << 8) >> 24).astype(jnp.float32) * sv
            f3 = (v >> 24).astype(jnp.float32) * sv
            stage_v[2 * j, rs] = _f16_bits(f0) | (_f16_bits(f1) << 16)
            stage_v[2 * j + 1, rs] = _f16_bits(f2) | (_f16_bits(f3) << 16)
        return 0

    lax.fori_loop(0, _BPW // 16, blk, 0)
    pltpu.sync_copy(stage_v, out_hbm.at[:, pl.ds(base, _BPW)])


_mesh = plsc.VectorSubcoreMesh(core_axis_name="c", subcore_axis_name="s")

_sc_lookup = functools.partial(
    pl.kernel,
    out_type=jax.ShapeDtypeStruct((_DIM // 2, _BATCH), jnp.int32),
    mesh=_mesh,
    scratch_types=[
        pltpu.VMEM((_BPW,), jnp.int32),
        pltpu.VMEM((_BPW, _DIM // 4), jnp.int32),
        pltpu.VMEM((_BPW,), jnp.int32),
        pltpu.VMEM((_DIM // 2, _BPW), jnp.int32),
        pltpu.SemaphoreType.DMA,
    ],
    compiler_params=pltpu.CompilerParams(
        needs_layout_passes=False, use_tc_tiling_on_sc=False),
)(_dequant_body)


def kernel(indices, weight, scales):
    idx = indices.astype(jnp.int32)
    w_packed = lax.bitcast_convert_type(
        weight.reshape(_VOCAB, _DIM // 4, 4), jnp.int32)
    s_packed = lax.bitcast_convert_type(scales, jnp.int32)
    out32 = _sc_lookup(idx, w_packed, s_packed)
    f = lax.bitcast_convert_type(out32, jnp.float16)
    return jnp.transpose(f, (1, 0, 2)).reshape(_BATCH, _DIM)


# speedup vs baseline: 1.8287x; 1.8114x over previous
import functools

import jax
import jax.numpy as jnp
from jax import lax
from jax.experimental import pallas as pl
from jax.experimental.pallas import tpu as pltpu
from jax.experimental.pallas import tpu_sc as plsc

_VOCAB = 1000000
_DIM = 64
_BATCH = 16384
_NC = 2
_NS = 16
_NW = _NC * _NS
_BPW = _BATCH // _NW
_CHUNK = 128
_NCHUNK = _BPW // _CHUNK


def _f16_bits(x):
    b = plsc.bitcast(x, jnp.int32)
    sign = lax.shift_right_logical(b, 16) & 0x8000
    mag = b & 0x7FFFFFFF
    rnd = mag + 0xFFF + (lax.shift_right_logical(mag, 13) & 1)
    h = lax.shift_right_logical(rnd, 13) - (112 << 10)
    return sign | jnp.maximum(h, 0)


def _dequant_body(idx_hbm, w_hbm, s_hbm, out_hbm,
                  idx_v, rows_v, sc_v, stage_v, sem):
    wid = lax.axis_index("s") * _NC + lax.axis_index("c")
    base = wid * _BPW

    pltpu.sync_copy(idx_hbm.at[pl.ds(base, _BPW)], idx_v)

    copies = []
    for q in range(_NCHUNK):
        sl = pl.ds(q * _CHUNK, _CHUNK)
        copies.append(pltpu.async_copy(s_hbm.at[idx_v.at[sl]], sc_v.at[sl], sem))

    def fire(c, _):
        iv = idx_v[pl.ds(c * 16, 16)]
        for r in range(16):
            pltpu.async_copy(w_hbm.at[pl.ds(iv[r], 1), :],
                             rows_v.at[pl.ds(c * 16 + r, 1), :], sem)
        return 0

    lax.fori_loop(0, _BPW // 16, fire, 0)

    def drain(r, _):
        pltpu.make_async_copy(w_hbm.at[pl.ds(0, 1), :],
                              rows_v.at[pl.ds(0, 1), :], sem).wait()
        return 0

    lax.fori_loop(0, _BPW, drain, 0)
    for c in copies:
        c.wait()

    lane = lax.iota(jnp.int32, 16)
    src_lane = lane >> 1
    lsh_e = 24 - 16 * (lane & 1)
    lsh_o = 16 - 16 * (lane & 1)
    dnums = lax.GatherDimensionNumbers(
        offset_dims=(), collapsed_slice_dims=(0,), start_index_map=(0,))

    def blk(t, _):
        sp16 = sc_v[pl.ds(t * 16, 16)]
        s0v = plsc.bitcast(((sp16 & 0x7FFF) << 13) + 0x38000000, jnp.float32)
        s1v = plsc.bitcast((((sp16 >> 16) & 0x7FFF) << 13) + 0x38000000,
                           jnp.float32)
        for r in range(16):
            i = t * 16 + r
            row32 = plsc.bitcast(rows_v[i], jnp.int32)
            s0 = jnp.broadcast_to(s0v[r], (16,))
            s1 = jnp.broadcast_to(s1v[r], (16,))
            bi = jnp.broadcast_to(i, (16,))
            for h, sv in ((0, s0), (1, s1)):
                g = lax.gather(row32, (src_lane + 8 * h)[:, None], dnums, (1,),
                               mode=lax.GatherScatterMode.PROMISE_IN_BOUNDS)
                e = ((g << lsh_e) >> 24).astype(jnp.float32) * sv
                o = ((g << lsh_o) >> 24).astype(jnp.float32) * sv
                word = _f16_bits(e) | (_f16_bits(o) << 16)
                plsc.store_scatter(stage_v, [16 * h + lane, bi], word)
        return 0

    lax.fori_loop(0, _BPW // 16, blk, 0)
    pltpu.sync_copy(stage_v, out_hbm.at[:, pl.ds(base, _BPW)])


_mesh = plsc.VectorSubcoreMesh(core_axis_name="c", subcore_axis_name="s")

_sc_lookup = functools.partial(
    pl.kernel,
    out_type=jax.ShapeDtypeStruct((_DIM // 2, _BATCH), jnp.int32),
    mesh=_mesh,
    scratch_types=[
        pltpu.VMEM((_BPW,), jnp.int32),
        pltpu.VMEM((_BPW, _DIM), jnp.int8),
        pltpu.VMEM((_BPW,), jnp.int32),
        pltpu.VMEM((_DIM // 2, _BPW), jnp.int32),
        pltpu.SemaphoreType.DMA,
    ],
    compiler_params=pltpu.CompilerParams(
        needs_layout_passes=False, use_tc_tiling_on_sc=False),
)(_dequant_body)


def kernel(indices, weight, scales):
    idx = indices.astype(jnp.int32)
    s_packed = lax.bitcast_convert_type(scales, jnp.int32)
    out32 = _sc_lookup(idx, weight, s_packed)
    f = lax.bitcast_convert_type(out32, jnp.float16)
    return jnp.transpose(f, (1, 0, 2)).reshape(_BATCH, _DIM)


# speedup vs baseline: 2.4889x; 1.3610x over previous
import functools

import jax
import jax.numpy as jnp
from jax import lax
from jax.experimental import pallas as pl
from jax.experimental.pallas import tpu as pltpu
from jax.experimental.pallas import tpu_sc as plsc

_VOCAB = 1000000
_DIM = 64
_BATCH = 16384
_NC = 2
_NS = 16
_NW = _NC * _NS
_BPW = _BATCH // _NW
_CHUNK = 128
_NCHUNK = _BPW // _CHUNK
_BCH = 64


def _f16_bits(x):
    b = plsc.bitcast(x, jnp.int32)
    sign = lax.shift_right_logical(b, 16) & 0x8000
    mag = b & 0x7FFFFFFF
    rnd = mag + 0xFFF + (lax.shift_right_logical(mag, 13) & 1)
    h = lax.shift_right_logical(rnd, 13) - (112 << 10)
    return sign | jnp.maximum(h, 0)


def _scales_body(idx_hbm, s_hbm, out_hbm, idx_v, sc_v, sem):
    wid = lax.axis_index("s") * _NC + lax.axis_index("c")
    base = wid * _BPW
    pltpu.sync_copy(idx_hbm.at[pl.ds(base, _BPW)], idx_v)
    copies = []
    for q in range(_NCHUNK):
        sl = pl.ds(q * _CHUNK, _CHUNK)
        copies.append(pltpu.async_copy(s_hbm.at[idx_v.at[sl]], sc_v.at[sl], sem))
    for c in copies:
        c.wait()
    pltpu.sync_copy(sc_v, out_hbm.at[pl.ds(base, _BPW)])


_sc_scales = functools.partial(
    pl.kernel,
    out_type=jax.ShapeDtypeStruct((_BATCH,), jnp.int32),
    mesh=plsc.VectorSubcoreMesh(core_axis_name="c", subcore_axis_name="s"),
    scratch_types=[
        pltpu.VMEM((_BPW,), jnp.int32),
        pltpu.VMEM((_BPW,), jnp.int32),
        pltpu.SemaphoreType.DMA,
    ],
    compiler_params=pltpu.CompilerParams(
        needs_layout_passes=False, use_tc_tiling_on_sc=False),
)(_scales_body)


def _dequant_body(idx_hbm, w_hbm, sg_hbm, out_hbm,
                  idx_v, blk_v, sc_v, stage_v, sem):
    wid = lax.axis_index("s") * _NC + lax.axis_index("c")
    base = pl.multiple_of(wid * _BPW, _BPW)

    pltpu.sync_copy(idx_hbm.at[pl.ds(base, _BPW)], idx_v)
    pltpu.sync_copy(sg_hbm.at[pl.ds(base, _BPW)], sc_v)

    lane = lax.iota(jnp.int32, 16)
    blk8 = blk_v.bitcast(jnp.int8)

    def chunk(c, _):
        c0 = c * _BCH

        def fire(b, _):
            iv = idx_v[pl.ds(c0 + b * 16, 16)]
            for r in range(16):
                ab = pl.multiple_of((iv[r] >> 5) << 5, 32)
                pltpu.async_copy(w_hbm.at[pl.ds(ab, 32), :],
                                 blk8.at[b * 16 + r], sem)
            return 0

        lax.fori_loop(0, _BCH // 16, fire, 0)

        def drain(r, _):
            pltpu.make_async_copy(w_hbm.at[pl.ds(0, 32), :],
                                  blk8.at[0], sem).wait()
            return 0

        lax.fori_loop(0, _BCH, drain, 0)

        def sub(b, _):
            i0 = c0 + b * 16
            rs = pl.ds(i0, 16)
            sp16 = sc_v[rs]
            s0v = plsc.bitcast(((sp16 & 0x7FFF) << 13) + 0x38000000,
                               jnp.float32)
            s1v = plsc.bitcast((((sp16 >> 16) & 0x7FFF) << 13) + 0x38000000,
                               jnp.float32)
            iv = idx_v[rs]
            svv = (iv >> 2) & 7
            lshv = 24 - ((iv & 3) << 3)
            islot = (b * 16) + lane
            for a in range(_DIM // 2):
                sv = s0v if a < 16 else s1v
                w0 = plsc.load_gather(blk_v, [islot, svv, lane * 0 + 2 * a])
                w1 = plsc.load_gather(blk_v, [islot, svv, lane * 0 + 2 * a + 1])
                v0 = ((w0 << lshv) >> 24).astype(jnp.float32) * sv
                v1 = ((w1 << lshv) >> 24).astype(jnp.float32) * sv
                stage_v[a, rs] = _f16_bits(v0) | (_f16_bits(v1) << 16)
            return 0

        lax.fori_loop(0, _BCH // 16, sub, 0)
        return 0

    lax.fori_loop(0, _BPW // _BCH, chunk, 0)
    pltpu.sync_copy(stage_v, out_hbm.at[:, pl.ds(base, _BPW)])


_sc_dequant = functools.partial(
    pl.kernel,
    out_type=jax.ShapeDtypeStruct((_DIM // 2, _BATCH), jnp.int32),
    mesh=plsc.VectorSubcoreMesh(core_axis_name="c", subcore_axis_name="s"),
    scratch_types=[
        pltpu.VMEM((_BPW,), jnp.int32),
        pltpu.VMEM((_BCH, 8, _DIM), jnp.int32),
        pltpu.VMEM((_BPW,), jnp.int32),
        pltpu.VMEM((_DIM // 2, _BPW), jnp.int32),
        pltpu.SemaphoreType.DMA,
    ],
    compiler_params=pltpu.CompilerParams(needs_layout_passes=False),
)(_dequant_body)


def kernel(indices, weight, scales):
    idx = indices.astype(jnp.int32)
    s_packed = lax.bitcast_convert_type(scales, jnp.int32)
    sg = _sc_scales(idx, s_packed)
    out32 = _sc_dequant(idx, weight, sg)
    f = lax.bitcast_convert_type(out32, jnp.float16)
    return jnp.transpose(f, (1, 0, 2)).reshape(_BATCH, _DIM)


# speedup vs baseline: 6.5347x; 2.6256x over previous
import functools

import jax
import jax.numpy as jnp
from jax import lax
from jax.experimental import pallas as pl
from jax.experimental.pallas import tpu as pltpu
from jax.experimental.pallas import tpu_sc as plsc

_VOCAB = 1000000
_DIM = 64
_BATCH = 16384
_NC = 2
_NS = 16
_NW = _NC * _NS
_BPW = _BATCH // _NW
_CHUNK = 128
_NCHUNK = _BPW // _CHUNK
_BCH = 32


def _f16_bits(x):
    b = plsc.bitcast(x, jnp.int32)
    sign = lax.shift_right_logical(b, 16) & 0x8000
    mag = b & 0x7FFFFFFF
    rnd = mag + 0xFFF + (lax.shift_right_logical(mag, 13) & 1)
    h = lax.shift_right_logical(rnd, 13) - (112 << 10)
    return sign | jnp.maximum(h, 0)


def _scales_body(idx_hbm, s_hbm, out_hbm, idx_v, sc_v, sem):
    wid = lax.axis_index("s") * _NC + lax.axis_index("c")
    base = wid * _BPW
    pltpu.sync_copy(idx_hbm.at[pl.ds(base, _BPW)], idx_v)
    copies = []
    for q in range(_NCHUNK):
        sl = pl.ds(q * _CHUNK, _CHUNK)
        copies.append(pltpu.async_copy(s_hbm.at[idx_v.at[sl]], sc_v.at[sl], sem))
    for c in copies:
        c.wait()
    pltpu.sync_copy(sc_v, out_hbm.at[pl.ds(base, _BPW)])


_sc_scales = functools.partial(
    pl.kernel,
    out_type=jax.ShapeDtypeStruct((_BATCH,), jnp.int32),
    mesh=plsc.VectorSubcoreMesh(core_axis_name="c", subcore_axis_name="s"),
    scratch_types=[
        pltpu.VMEM((_BPW,), jnp.int32),
        pltpu.VMEM((_BPW,), jnp.int32),
        pltpu.SemaphoreType.DMA,
    ],
    compiler_params=pltpu.CompilerParams(
        needs_layout_passes=False, use_tc_tiling_on_sc=False),
)(_scales_body)


def _dequant_body(idx_hbm, wt_hbm, sg_hbm, out_hbm,
                  idx_v, blk_v, sc_v, stage_v, sem):
    wid = lax.axis_index("s") * _NC + lax.axis_index("c")
    base = pl.multiple_of(wid * _BPW, _BPW)

    pltpu.sync_copy(idx_hbm.at[pl.ds(base, _BPW)], idx_v)
    pltpu.sync_copy(sg_hbm.at[pl.ds(base, _BPW)], sc_v)

    lane = lax.iota(jnp.int32, 16)
    blk8 = blk_v.bitcast(jnp.int8)

    def chunk(c, _):
        c0 = c * _BCH

        def fire(b, _):
            iv = idx_v[pl.ds(c0 + b * 16, 16)]
            for r in range(16):
                ab = pl.multiple_of((iv[r] >> 7) << 7, 128)
                pltpu.async_copy(wt_hbm.at[:, pl.ds(ab, 128)],
                                 blk8.at[b * 16 + r], sem)
            return 0

        lax.fori_loop(0, _BCH // 16, fire, 0)

        def drain(r, _):
            pltpu.make_async_copy(wt_hbm.at[:, pl.ds(0, 128)],
                                  blk8.at[0], sem).wait()
            return 0

        lax.fori_loop(0, _BCH, drain, 0)

        def sub(b, _):
            i0 = c0 + b * 16
            rs = pl.ds(i0, 16)
            sp16 = sc_v[rs]
            s0v = plsc.bitcast(((sp16 & 0x7FFF) << 13) + 0x38000000,
                               jnp.float32)
            s1v = plsc.bitcast((((sp16 >> 16) & 0x7FFF) << 13) + 0x38000000,
                               jnp.float32)
            iv = idx_v[rs]
            lanev = iv & 127
            islot = (b * 16) + lane
            for j in range(16):
                sv = s0v if j < 8 else s1v
                v = plsc.load_gather(blk_v, [islot, lane * 0 + j, lanev])
                f0 = ((v << 24) >> 24).astype(jnp.float32) * sv
                f1 = ((v << 16) >> 24).astype(jnp.float32) * sv
                f2 = ((v << 8) >> 24).astype(jnp.float32) * sv
                f3 = (v >> 24).astype(jnp.float32) * sv
                stage_v[2 * j, rs] = _f16_bits(f0) | (_f16_bits(f1) << 16)
                stage_v[2 * j + 1, rs] = _f16_bits(f2) | (_f16_bits(f3) << 16)
            return 0

        lax.fori_loop(0, _BCH // 16, sub, 0)
        return 0

    lax.fori_loop(0, _BPW // _BCH, chunk, 0)
    pltpu.sync_copy(stage_v, out_hbm.at[:, pl.ds(base, _BPW)])


_sc_dequant = functools.partial(
    pl.kernel,
    out_type=jax.ShapeDtypeStruct((_DIM // 2, _BATCH), jnp.int32),
    mesh=plsc.VectorSubcoreMesh(core_axis_name="c", subcore_axis_name="s"),
    scratch_types=[
        pltpu.VMEM((_BPW,), jnp.int32),
        pltpu.VMEM((_BCH, 16, 128), jnp.int32),
        pltpu.VMEM((_BPW,), jnp.int32),
        pltpu.VMEM((_DIM // 2, _BPW), jnp.int32),
        pltpu.SemaphoreType.DMA,
    ],
    compiler_params=pltpu.CompilerParams(needs_layout_passes=False),
)(_dequant_body)


def kernel(indices, weight, scales):
    idx = indices.astype(jnp.int32)
    s_packed = lax.bitcast_convert_type(scales, jnp.int32)
    sg = _sc_scales(idx, s_packed)
    out32 = _sc_dequant(idx, weight.T, sg)
    f = lax.bitcast_convert_type(out32, jnp.float16)
    return jnp.transpose(f, (1, 0, 2)).reshape(_BATCH, _DIM)


# speedup vs baseline: 6.7646x; 1.0352x over previous
import functools

import jax
import jax.numpy as jnp
from jax import lax
from jax.experimental import pallas as pl
from jax.experimental.pallas import tpu as pltpu
from jax.experimental.pallas import tpu_sc as plsc

_VOCAB = 1000000
_DIM = 64
_BATCH = 16384
_NC = 2
_NS = 16
_NW = _NC * _NS
_BPW = _BATCH // _NW
_CHUNK = 128
_NCHUNK = _BPW // _CHUNK
_BCH = 32


def _f16_bits(x):
    b = plsc.bitcast(x, jnp.int32)
    sign = lax.shift_right_logical(b, 16) & 0x8000
    mag = b & 0x7FFFFFFF
    rnd = mag + 0xFFF + (lax.shift_right_logical(mag, 13) & 1)
    h = lax.shift_right_logical(rnd, 13) - (112 << 10)
    return sign | jnp.maximum(h, 0)


def _scales_body(idx_hbm, s_hbm, out_hbm, idx_v, sc_v, sem):
    wid = lax.axis_index("s") * _NC + lax.axis_index("c")
    base = wid * _BPW
    pltpu.sync_copy(idx_hbm.at[pl.ds(base, _BPW)], idx_v)
    copies = []
    for q in range(_NCHUNK):
        sl = pl.ds(q * _CHUNK, _CHUNK)
        copies.append(pltpu.async_copy(s_hbm.at[idx_v.at[sl]], sc_v.at[sl], sem))
    for c in copies:
        c.wait()
    pltpu.sync_copy(sc_v, out_hbm.at[pl.ds(base, _BPW)])


_sc_scales = functools.partial(
    pl.kernel,
    out_type=jax.ShapeDtypeStruct((_BATCH,), jnp.int32),
    mesh=plsc.VectorSubcoreMesh(core_axis_name="c", subcore_axis_name="s"),
    scratch_types=[
        pltpu.VMEM((_BPW,), jnp.int32),
        pltpu.VMEM((_BPW,), jnp.int32),
        pltpu.SemaphoreType.DMA,
    ],
    compiler_params=pltpu.CompilerParams(
        needs_layout_passes=False, use_tc_tiling_on_sc=False),
)(_scales_body)


def _dequant_body(idx_hbm, wt_hbm, sg_hbm, out_hbm,
                  idx_v, blka_v, blkb_v, sc_v, stage_v, sema, semb):
    wid = lax.axis_index("s") * _NC + lax.axis_index("c")
    base = pl.multiple_of(wid * _BPW, _BPW)

    pltpu.sync_copy(idx_hbm.at[pl.ds(base, _BPW)], idx_v)
    pltpu.sync_copy(sg_hbm.at[pl.ds(base, _BPW)], sc_v)

    lane = lax.iota(jnp.int32, 16)
    blka8 = blka_v.bitcast(jnp.int8)
    blkb8 = blkb_v.bitcast(jnp.int8)

    def fire(buf8, sem, c):
        iv = idx_v[pl.ds(c * 16, 16)]
        for r in range(16):
            ab = pl.multiple_of((iv[r] >> 7) << 7, 128)
            pltpu.async_copy(wt_hbm.at[:, pl.ds(ab, 128)], buf8.at[r], sem)

    def drain(buf8, sem):
        for _ in range(16):
            pltpu.make_async_copy(wt_hbm.at[:, pl.ds(0, 128)],
                                  buf8.at[0], sem).wait()

    def compute(buf, c):
        i0 = c * 16
        rs = pl.ds(i0, 16)
        sp16 = sc_v[rs]
        s0v = plsc.bitcast(((sp16 & 0x7FFF) << 13) + 0x38000000, jnp.float32)
        s1v = plsc.bitcast((((sp16 >> 16) & 0x7FFF) << 13) + 0x38000000,
                           jnp.float32)
        lanev = idx_v[rs] & 127
        for j in range(16):
            sv = s0v if j < 8 else s1v
            v = plsc.load_gather(buf, [lane, lane * 0 + j, lanev])
            f0 = ((v << 24) >> 24).astype(jnp.float32) * sv
            f1 = ((v << 16) >> 24).astype(jnp.float32) * sv
            f2 = ((v << 8) >> 24).astype(jnp.float32) * sv
            f3 = (v >> 24).astype(jnp.float32) * sv
            stage_v[2 * j, rs] = _f16_bits(f0) | (_f16_bits(f1) << 16)
            stage_v[2 * j + 1, rs] = _f16_bits(f2) | (_f16_bits(f3) << 16)

    fire(blka8, sema, 0)

    def body(i, _):
        fire(blkb8, semb, 2 * i + 1)
        drain(blka8, sema)
        compute(blka_v, 2 * i)

        @pl.when(i < _BPW // 32 - 1)
        def _():
            fire(blka8, sema, 2 * i + 2)

        drain(blkb8, semb)
        compute(blkb_v, 2 * i + 1)
        return 0

    lax.fori_loop(0, _BPW // 32, body, 0)
    pltpu.sync_copy(stage_v, out_hbm.at[:, pl.ds(base, _BPW)])


_sc_dequant = functools.partial(
    pl.kernel,
    out_type=jax.ShapeDtypeStruct((_DIM // 2, _BATCH), jnp.int32),
    mesh=plsc.VectorSubcoreMesh(core_axis_name="c", subcore_axis_name="s"),
    scratch_types=[
        pltpu.VMEM((_BPW,), jnp.int32),
        pltpu.VMEM((16, 16, 128), jnp.int32),
        pltpu.VMEM((16, 16, 128), jnp.int32),
        pltpu.VMEM((_BPW,), jnp.int32),
        pltpu.VMEM((_DIM // 2, _BPW), jnp.int32),
        pltpu.SemaphoreType.DMA,
        pltpu.SemaphoreType.DMA,
    ],
    compiler_params=pltpu.CompilerParams(needs_layout_passes=False),
)(_dequant_body)


def kernel(indices, weight, scales):
    idx = indices.astype(jnp.int32)
    s_packed = lax.bitcast_convert_type(scales, jnp.int32)
    sg = _sc_scales(idx, s_packed)
    out32 = _sc_dequant(idx, weight.T, sg)
    f = lax.bitcast_convert_type(out32, jnp.float16)
    return jnp.transpose(f, (1, 0, 2)).reshape(_BATCH, _DIM)


# speedup vs baseline: 9.0531x; 1.3383x over previous
import functools

import jax
import jax.numpy as jnp
from jax import lax
from jax.experimental import pallas as pl
from jax.experimental.pallas import tpu as pltpu
from jax.experimental.pallas import tpu_sc as plsc

_VOCAB = 1000000
_DIM = 64
_BATCH = 16384
_NC = 2
_NS = 16
_NW = _NC * _NS
_BPW = _BATCH // _NW
_CHUNK = 128
_NCHUNK = _BPW // _CHUNK
_BCH = 32


def _f16_bits(x):
    b = plsc.bitcast(x, jnp.int32)
    sign = lax.shift_right_logical(b, 16) & 0x8000
    mag = b & 0x7FFFFFFF
    rnd = mag + 0xFFF + (lax.shift_right_logical(mag, 13) & 1)
    h = lax.shift_right_logical(rnd, 13) - (112 << 10)
    return sign | jnp.maximum(h, 0)


def _scales_body(idx_hbm, s_hbm, out_hbm, idx_v, sc_v, sem):
    wid = lax.axis_index("s") * _NC + lax.axis_index("c")
    base = wid * _BPW
    pltpu.sync_copy(idx_hbm.at[pl.ds(base, _BPW)], idx_v)
    copies = []
    for q in range(_NCHUNK):
        sl = pl.ds(q * _CHUNK, _CHUNK)
        copies.append(pltpu.async_copy(s_hbm.at[idx_v.at[sl]], sc_v.at[sl], sem))
    for c in copies:
        c.wait()
    pltpu.sync_copy(sc_v, out_hbm.at[pl.ds(base, _BPW)])


_sc_scales = functools.partial(
    pl.kernel,
    out_type=jax.ShapeDtypeStruct((_BATCH,), jnp.int32),
    mesh=plsc.VectorSubcoreMesh(core_axis_name="c", subcore_axis_name="s"),
    scratch_types=[
        pltpu.VMEM((_BPW,), jnp.int32),
        pltpu.VMEM((_BPW,), jnp.int32),
        pltpu.SemaphoreType.DMA,
    ],
    compiler_params=pltpu.CompilerParams(
        needs_layout_passes=False, use_tc_tiling_on_sc=False),
)(_scales_body)


def _dequant_body(idx_hbm, wt_hbm, st_hbm, out_hbm,
                  idx_v, blka_v, blkb_v, sca_v, scb_v, stage_v, sema, semb):
    wid = lax.axis_index("s") * _NC + lax.axis_index("c")
    base = pl.multiple_of(wid * _BPW, _BPW)

    pltpu.sync_copy(idx_hbm.at[pl.ds(base, _BPW)], idx_v)

    lane = lax.iota(jnp.int32, 16)
    blka8 = blka_v.bitcast(jnp.int8)
    blkb8 = blkb_v.bitcast(jnp.int8)
    sca16 = sca_v.bitcast(jnp.float16)
    scb16 = scb_v.bitcast(jnp.float16)

    def fire(buf8, scb, sem, c):
        iv = idx_v[pl.ds(c * 16, 16)]
        for r in range(16):
            ab = pl.multiple_of((iv[r] >> 7) << 7, 128)
            pltpu.async_copy(wt_hbm.at[:, pl.ds(ab, 128)], buf8.at[r], sem)
            pltpu.async_copy(st_hbm.at[:, pl.ds(ab, 128)], scb.at[r], sem)

    def drain(buf8, scb, sem):
        for _ in range(16):
            pltpu.make_async_copy(wt_hbm.at[:, pl.ds(0, 128)],
                                  buf8.at[0], sem).wait()
            pltpu.make_async_copy(st_hbm.at[:, pl.ds(0, 128)],
                                  scb.at[0], sem).wait()

    def compute(buf, scv, c):
        i0 = c * 16
        rs = pl.ds(i0, 16)
        lanev = idx_v[rs] & 127
        sp16 = plsc.load_gather(scv, [lane, lane * 0, lanev])
        s0v = plsc.bitcast(((sp16 & 0x7FFF) << 13) + 0x38000000, jnp.float32)
        s1v = plsc.bitcast((((sp16 >> 16) & 0x7FFF) << 13) + 0x38000000,
                           jnp.float32)
        for j in range(16):
            sv = s0v if j < 8 else s1v
            v = plsc.load_gather(buf, [lane, lane * 0 + j, lanev])
            f0 = ((v << 24) >> 24).astype(jnp.float32) * sv
            f1 = ((v << 16) >> 24).astype(jnp.float32) * sv
            f2 = ((v << 8) >> 24).astype(jnp.float32) * sv
            f3 = (v >> 24).astype(jnp.float32) * sv
            stage_v[2 * j, rs] = _f16_bits(f0) | (_f16_bits(f1) << 16)
            stage_v[2 * j + 1, rs] = _f16_bits(f2) | (_f16_bits(f3) << 16)

    fire(blka8, sca16, sema, 0)

    def body(i, _):
        fire(blkb8, scb16, semb, 2 * i + 1)
        drain(blka8, sca16, sema)
        compute(blka_v, sca_v, 2 * i)

        @pl.when(i < _BPW // 32 - 1)
        def _():
            fire(blka8, sca16, sema, 2 * i + 2)

        drain(blkb8, scb16, semb)
        compute(blkb_v, scb_v, 2 * i + 1)
        return 0

    lax.fori_loop(0, _BPW // 32, body, 0)
    pltpu.sync_copy(stage_v, out_hbm.at[:, pl.ds(base, _BPW)])


_sc_dequant = functools.partial(
    pl.kernel,
    out_type=jax.ShapeDtypeStruct((_DIM // 2, _BATCH), jnp.int32),
    mesh=plsc.VectorSubcoreMesh(core_axis_name="c", subcore_axis_name="s"),
    scratch_types=[
        pltpu.VMEM((_BPW,), jnp.int32),
        pltpu.VMEM((16, 16, 128), jnp.int32),
        pltpu.VMEM((16, 16, 128), jnp.int32),
        pltpu.VMEM((16, 1, 128), jnp.int32),
        pltpu.VMEM((16, 1, 128), jnp.int32),
        pltpu.VMEM((_DIM // 2, _BPW), jnp.int32),
        pltpu.SemaphoreType.DMA,
        pltpu.SemaphoreType.DMA,
    ],
    compiler_params=pltpu.CompilerParams(needs_layout_passes=False),
)(_dequant_body)


def kernel(indices, weight, scales):
    idx = indices.astype(jnp.int32)
    out32 = _sc_dequant(idx, weight.T, scales.T)
    f = lax.bitcast_convert_type(out32, jnp.float16)
    return jnp.transpose(f, (1, 0, 2)).reshape(_BATCH, _DIM)
